# Initial kernel scaffold; baseline (speedup 1.0000x reference)
import jax
import jax.numpy as jnp
from jax.experimental import pallas as pl


def kernel(x, edge_index, W1a, b1a, W2a, b2a, W1m, b1m, W2m, b2m, W1s, b1s, W2s, b2s):
    raise NotImplementedError("write your pallas kernel here")


# speedup vs baseline: 9.8706x; 9.8706x over previous
import functools

import jax
import jax.numpy as jnp
from jax import lax
from jax.experimental import pallas as pl
from jax.experimental.pallas import tpu as pltpu
from jax.experimental.pallas import tpu_sc as plsc

N_NODES = 10000
E_EDGES = 320000
D_IN = 128
D_OUT = 16
D_HID = 32

NCORE = 2
NSUB = 16
NW = NCORE * NSUB
CHUNK = 128
EPW = -(-E_EDGES // NW)
NCHUNK = -(-EPW // CHUNK)
EPW_PAD = NCHUNK * CHUNK
N_PAD = NSUB * 640
TILE_ROWS = N_PAD // NSUB
ZROWS = 64

ROW_BLK = 1000


@functools.lru_cache(maxsize=None)
def _make_agg(feat):
    mesh = plsc.VectorSubcoreMesh(core_axis_name="c", subcore_axis_name="s",
                                  num_cores=NCORE, num_subcores=NSUB)

    @functools.partial(
        pl.kernel,
        out_type=jax.ShapeDtypeStruct((NCORE, N_PAD, feat), jnp.float32),
        mesh=mesh,
        scratch_types=[
            pltpu.VMEM((NCHUNK, CHUNK), jnp.int32),
            pltpu.VMEM((NCHUNK, CHUNK), jnp.int32),
            pltpu.VMEM((2, CHUNK, feat), jnp.float32),
            pltpu.VMEM((ZROWS, feat), jnp.float32),
            pltpu.VMEM_SHARED((N_PAD, feat), jnp.float32),
            pltpu.SemaphoreType.DMA,
            pltpu.SemaphoreType.DMA,
        ],
        compiler_params=pltpu.CompilerParams(use_tc_tiling_on_sc=False),
    )
    def agg(v_hbm, src_hbm, dst_hbm, out_hbm, src_v, dst_v, rows_v, zero_v,
            acc, gsem0, gsem1):
        cid = lax.axis_index("c")
        sid = lax.axis_index("s")
        wid = cid * NSUB + sid

        pltpu.sync_copy(src_hbm.at[wid], src_v)
        pltpu.sync_copy(dst_hbm.at[wid], dst_v)

        zero16 = jnp.zeros((16,), jnp.float32)
        for r in range(ZROWS):
            for q in range(feat // 16):
                zero_v[r, pl.ds(q * 16, 16)] = zero16
        base = sid * TILE_ROWS
        for k in range(TILE_ROWS // ZROWS):
            pltpu.sync_copy(zero_v, acc.at[pl.ds(base + k * ZROWS, ZROWS)])
        plsc.subcore_barrier()

        def body(j, carry):
            pltpu.async_copy(v_hbm.at[src_v.at[j]], rows_v.at[0], gsem0).wait()
            pltpu.sync_copy(rows_v.at[0], acc.at[dst_v.at[j]], add=True)
            return carry

        lax.fori_loop(0, NCHUNK, body, 0)
        plsc.subcore_barrier()
        pltpu.sync_copy(acc.at[pl.ds(base, TILE_ROWS)],
                        out_hbm.at[cid, pl.ds(base, TILE_ROWS)])

    return agg


def _tc_proj(x, w):
    def body(x_ref, w_ref, o_ref):
        o_ref[...] = jnp.dot(x_ref[...], w_ref[...],
                             preferred_element_type=jnp.float32)

    return pl.pallas_call(
        body,
        grid=(N_NODES // ROW_BLK,),
        in_specs=[pl.BlockSpec((ROW_BLK, D_IN), lambda i: (i, 0)),
                  pl.BlockSpec((D_IN, D_HID), lambda i: (0, 0))],
        out_specs=pl.BlockSpec((ROW_BLK, D_HID), lambda i: (i, 0)),
        out_shape=jax.ShapeDtypeStruct((N_NODES, D_HID), jnp.float32),
    )(x, w)


def _tc_mlp1(y, a0, a1, b1, w2, b2):
    def body(y_ref, a0_ref, a1_ref, b1_ref, w2_ref, b2_ref, o_ref):
        g = y_ref[...] + a0_ref[...] + a1_ref[...] + b1_ref[...]
        h1 = jnp.maximum(g, 0.0)
        h = jnp.dot(h1, w2_ref[...], preferred_element_type=jnp.float32)
        o_ref[...] = jnp.maximum(h + b2_ref[...], 0.0)

    nb = N_NODES // ROW_BLK
    return pl.pallas_call(
        body,
        grid=(nb,),
        in_specs=[pl.BlockSpec((ROW_BLK, D_HID), lambda i: (i, 0)),
                  pl.BlockSpec((ROW_BLK, D_HID), lambda i: (i, 0)),
                  pl.BlockSpec((ROW_BLK, D_HID), lambda i: (i, 0)),
                  pl.BlockSpec((1, D_HID), lambda i: (0, 0)),
                  pl.BlockSpec((D_HID, D_OUT), lambda i: (0, 0)),
                  pl.BlockSpec((1, D_OUT), lambda i: (0, 0))],
        out_specs=pl.BlockSpec((ROW_BLK, D_OUT), lambda i: (i, 0)),
        out_shape=jax.ShapeDtypeStruct((N_NODES, D_OUT), jnp.float32),
    )(y, a0, a1, b1, w2, b2)


def _tc_mlp2(h, a0, a1, w1m, b1m, w2m, b2m, w1s, b1s, w2s, b2s):
    def body(h_ref, a0_ref, a1_ref, w1m_ref, b1m_ref, w2m_ref, b2m_ref,
             w1s_ref, b1s_ref, w2s_ref, b2s_ref, mu_ref, ls_ref):
        g = h_ref[...] + a0_ref[...] + a1_ref[...]
        tm = jnp.dot(g, w1m_ref[...], preferred_element_type=jnp.float32)
        tm = jnp.maximum(tm + b1m_ref[...], 0.0)
        mu_ref[...] = jnp.dot(tm, w2m_ref[...],
                              preferred_element_type=jnp.float32) + b2m_ref[...]
        ts = jnp.dot(g, w1s_ref[...], preferred_element_type=jnp.float32)
        ts = jnp.maximum(ts + b1s_ref[...], 0.0)
        ls_ref[...] = jnp.dot(ts, w2s_ref[...],
                              preferred_element_type=jnp.float32) + b2s_ref[...]

    nb = N_NODES // ROW_BLK
    row = lambda i: (i, 0)
    fix = lambda i: (0, 0)
    return pl.pallas_call(
        body,
        grid=(nb,),
        in_specs=[pl.BlockSpec((ROW_BLK, D_OUT), row),
                  pl.BlockSpec((ROW_BLK, D_OUT), row),
                  pl.BlockSpec((ROW_BLK, D_OUT), row),
                  pl.BlockSpec((D_OUT, D_HID), fix),
                  pl.BlockSpec((1, D_HID), fix),
                  pl.BlockSpec((D_HID, D_OUT), fix),
                  pl.BlockSpec((1, D_OUT), fix),
                  pl.BlockSpec((D_OUT, D_HID), fix),
                  pl.BlockSpec((1, D_HID), fix),
                  pl.BlockSpec((D_HID, D_OUT), fix),
                  pl.BlockSpec((1, D_OUT), fix)],
        out_specs=[pl.BlockSpec((ROW_BLK, D_OUT), row),
                   pl.BlockSpec((ROW_BLK, D_OUT), row)],
        out_shape=[jax.ShapeDtypeStruct((N_NODES, D_OUT), jnp.float32),
                   jax.ShapeDtypeStruct((N_NODES, D_OUT), jnp.float32)],
    )(h, a0, a1, w1m, b1m, w2m, b2m, w1s, b1s, w2s, b2s)


def kernel(x, edge_index, W1a, b1a, W2a, b2a, W1m, b1m, W2m, b2m,
           W1s, b1s, W2s, b2s):
    pad = NW * EPW_PAD - E_EDGES
    src = jnp.concatenate(
        [edge_index[0], jnp.zeros((pad,), jnp.int32)]).reshape(NW, NCHUNK, CHUNK)
    dst = jnp.concatenate(
        [edge_index[1], jnp.full((pad,), N_NODES, jnp.int32)]).reshape(
            NW, NCHUNK, CHUNK)

    y = _tc_proj(x, W1a)
    p = _make_agg(D_HID)(y, src, dst)
    h = _tc_mlp1(y, p[0, :N_NODES], p[1, :N_NODES],
                 b1a.reshape(1, -1), W2a, b2a.reshape(1, -1))
    q = _make_agg(D_OUT)(h, src, dst)
    mu, logstd = _tc_mlp2(h, q[0, :N_NODES], q[1, :N_NODES],
                          W1m, b1m.reshape(1, -1), W2m, b2m.reshape(1, -1),
                          W1s, b1s.reshape(1, -1), W2s, b2s.reshape(1, -1))
    return (mu, logstd)
